# baseline (speedup 1.0000x reference)
import functools

import jax
import jax.numpy as jnp
from jax import lax
from jax.experimental import pallas as pl
from jax.experimental.pallas import tpu as pltpu
from jax.experimental.pallas import tpu_sc as plsc

_F32 = jnp.float32

_NC = 2
_NS = 16
_NW = _NC * _NS
_SUB = 128


def _pq_body(x_ref, wi_ref, wj_ref, b1_ref, p_ref, q_ref):
    xv = x_ref[...]
    p_ref[...] = (
        jnp.dot(xv, wi_ref[...], preferred_element_type=_F32) + b1_ref[...]
    )
    q_ref[...] = jnp.dot(xv, wj_ref[...], preferred_element_type=_F32)


def _pq_call(x, wi, wj, b1row, bn):
    n, d = x.shape
    h = wi.shape[1]
    grid = n // bn
    return pl.pallas_call(
        _pq_body,
        grid=(grid,),
        in_specs=[
            pl.BlockSpec((bn, d), lambda i: (i, 0)),
            pl.BlockSpec((d, h), lambda i: (0, 0)),
            pl.BlockSpec((d, h), lambda i: (0, 0)),
            pl.BlockSpec((1, h), lambda i: (0, 0)),
        ],
        out_specs=[
            pl.BlockSpec((bn, h), lambda i: (i, 0)),
            pl.BlockSpec((bn, h), lambda i: (i, 0)),
        ],
        out_shape=[
            jax.ShapeDtypeStruct((n, h), _F32),
            jax.ShapeDtypeStruct((n, h), _F32),
        ],
    )(x, wi, wj, b1row)


def _gather_call(p, q, ei4p, nsub):
    n, h = p.shape
    nsub_pad = ei4p.shape[1]
    cpw = nsub_pad // _NW
    mesh = plsc.VectorSubcoreMesh(core_axis_name="c", subcore_axis_name="s")

    @functools.partial(
        pl.kernel,
        out_type=jax.ShapeDtypeStruct((nsub, _SUB, h // 2), jnp.int32),
        mesh=mesh,
        scratch_types=[
            pltpu.VMEM((cpw, 1, _SUB), jnp.int32),
            pltpu.VMEM((cpw, 1, _SUB), jnp.int32),
            pltpu.VMEM((_SUB, h), _F32),
            pltpu.VMEM((_SUB, h), _F32),
            pltpu.VMEM((_SUB, h), _F32),
            pltpu.VMEM((_SUB, h), _F32),
            pltpu.VMEM((_SUB, h // 2), jnp.int32),
            pltpu.VMEM((_SUB, h // 2), jnp.int32),
            pltpu.SemaphoreType.DMA,
            pltpu.SemaphoreType.DMA,
        ],
        compiler_params=pltpu.CompilerParams(needs_layout_passes=False),
    )
    def gather_k(
        p_hbm, q_hbm, ei_hbm, g_hbm, idxd_all, idxs_all, bufp0, bufq0,
        bufp1, bufq1, bufg0, bufg1, sem0, sem1
    ):
        wid = lax.axis_index("s") * _NC + lax.axis_index("c")
        w0 = wid * cpw
        nm = jnp.minimum(cpw, nsub - w0)

        pltpu.sync_copy(ei_hbm.at[1, pl.ds(w0, cpw)], idxd_all)
        pltpu.sync_copy(ei_hbm.at[0, pl.ds(w0, cpw)], idxs_all)

        bufs = ((bufp0, bufq0, bufg0, sem0), (bufp1, bufq1, bufg1, sem1))

        def issue(c, bp, bq, bg, sem):
            @pl.when(c < nm)
            def _():
                pltpu.async_copy(p_hbm.at[idxd_all.at[c, 0]], bp, sem)
                pltpu.async_copy(q_hbm.at[idxs_all.at[c, 0]], bq, sem)

        def process(c, bp, bq, bg, sem):
            @pl.when(c < nm)
            def _():
                pltpu.make_async_copy(
                    p_hbm.at[idxd_all.at[c, 0]], bp, sem
                ).wait()
                pltpu.make_async_copy(
                    q_hbm.at[idxs_all.at[c, 0]], bq, sem
                ).wait()

                def add_row(i, acc):
                    for j in range(h // 32):
                        sa = pl.ds(j * 32, 16)
                        sb = pl.ds(j * 32 + 16, 16)
                        va = bp[i, sa] + bq[i, sa]
                        vb = bp[i, sb] + bq[i, sb]
                        packed = plsc.pack(
                            va, vb, format=plsc.PackFormat.INTERLEAVED
                        )
                        bg[i, pl.ds(j * 16, 16)] = plsc.bitcast(
                            packed, jnp.int32
                        )
                    return acc

                lax.fori_loop(0, _SUB, add_row, 0)
                pltpu.sync_copy(bg, g_hbm.at[w0 + c])

        issue(0, *bufs[0])
        issue(1, *bufs[1])

        def body(k, carry):
            c0 = k * 2
            for b in range(2):
                bp, bq, bg, sem = bufs[b]
                process(c0 + b, bp, bq, bg, sem)
                issue(c0 + b + 2, bp, bq, bg, sem)
            return carry

        lax.fori_loop(0, (cpw + 1) // 2, body, 0)

    return gather_k(p, q, ei4p)


def _edge_body(g_ref, ea_ref, we_ref, w2_ref, b2_ref, a1_ref, a2_ref, o_ref):
    w = g_ref[...]
    lo = lax.bitcast_convert_type(w << 16, _F32)
    hi = lax.bitcast_convert_type(w & jnp.int32(-65536), _F32)
    z = jnp.concatenate([lo, hi], axis=1) + jnp.dot(
        ea_ref[...], we_ref[...], preferred_element_type=_F32
    )
    m1 = jnp.where(z >= 0.0, z, a1_ref[...] * z)
    y = jnp.dot(m1, w2_ref[...], preferred_element_type=_F32) + b2_ref[...]
    o_ref[...] = jnp.where(y >= 0.0, y, a2_ref[...] * y)


def _edge_call(g, ea, we, w2, b2row, a1row, a2row, be):
    e, h2 = g.shape
    h = 2 * h2
    ed = ea.shape[1]
    grid = e // be
    return pl.pallas_call(
        _edge_body,
        grid=(grid,),
        in_specs=[
            pl.BlockSpec((be, h2), lambda i: (i, 0)),
            pl.BlockSpec((be, ed), lambda i: (i, 0)),
            pl.BlockSpec((ed, h), lambda i: (0, 0)),
            pl.BlockSpec((h, h), lambda i: (0, 0)),
            pl.BlockSpec((1, h), lambda i: (0, 0)),
            pl.BlockSpec((1, h), lambda i: (0, 0)),
            pl.BlockSpec((1, h), lambda i: (0, 0)),
        ],
        out_specs=pl.BlockSpec((be, h), lambda i: (i, 0)),
        out_shape=jax.ShapeDtypeStruct((e, h), _F32),
    )(g, ea, we, w2, b2row, a1row, a2row)


def _scatter_call(m3, ei4p, n):
    nsub, sub, h = m3.shape
    nsub_pad = ei4p.shape[1]
    cpw = nsub_pad // _NW
    n_chunks = -(-n // _SUB)
    n_pad = n_chunks * _SUB
    cpt = -(-n_chunks // _NS)
    mesh = plsc.VectorSubcoreMesh(core_axis_name="c", subcore_axis_name="s")

    @functools.partial(
        pl.kernel,
        out_type=jax.ShapeDtypeStruct((_NC, n_chunks, _SUB, h), _F32),
        mesh=mesh,
        scratch_types=[
            pltpu.VMEM((cpw, 1, _SUB), jnp.int32),
            pltpu.VMEM((_SUB, h), _F32),
            pltpu.VMEM((_SUB, h), _F32),
            pltpu.VMEM_SHARED((n_pad, h), _F32),
            pltpu.SemaphoreType.DMA,
            pltpu.SemaphoreType.DMA,
        ],
    )
    def scatter_k(m_hbm, ei_hbm, out_hbm, idxd_all, bufm0, bufm1, acc,
                  sem0, sem1):
        cid = lax.axis_index("c")
        sid = lax.axis_index("s")
        wid = sid * _NC + cid
        w0 = wid * cpw
        nm = jnp.minimum(cpw, nsub - w0)

        pltpu.sync_copy(ei_hbm.at[1, pl.ds(w0, cpw)], idxd_all)

        def zrow(i, acc_c):
            for j in range(h // 16):
                bufm0[i, pl.ds(j * 16, 16)] = jnp.zeros((16,), _F32)
            return acc_c

        lax.fori_loop(0, _SUB, zrow, 0)
        for c in range(cpt):
            ck = c * _NS + sid

            @pl.when(ck < n_chunks)
            def _():
                row0 = pl.multiple_of(ck * _SUB, 8)
                pltpu.sync_copy(bufm0, acc.at[pl.ds(row0, _SUB)])

        plsc.subcore_barrier()

        bufs = ((bufm0, sem0), (bufm1, sem1))

        def issue(c, bm, sem):
            @pl.when(c < nm)
            def _():
                pltpu.async_copy(m_hbm.at[w0 + c], bm, sem)

        def process(c, bm, sem):
            @pl.when(c < nm)
            def _():
                pltpu.make_async_copy(m_hbm.at[w0 + c], bm, sem).wait()
                pltpu.sync_copy(bm, acc.at[idxd_all.at[c, 0]], add=True)

        issue(0, *bufs[0])
        issue(1, *bufs[1])

        def body(k, carry):
            c0 = k * 2
            for b in range(2):
                bm, sem = bufs[b]
                process(c0 + b, bm, sem)
                issue(c0 + b + 2, bm, sem)
            return carry

        lax.fori_loop(0, (cpw + 1) // 2, body, 0)
        plsc.subcore_barrier()

        for c in range(cpt):
            ck = c * _NS + sid

            @pl.when(ck < n_chunks)
            def _():
                row0 = pl.multiple_of(ck * _SUB, 8)
                pltpu.sync_copy(acc.at[pl.ds(row0, _SUB)], bufm0)
                pltpu.sync_copy(bufm0, out_hbm.at[cid, ck])

    return scatter_k(m3, ei4p)


def _node_body(
    k, x_ref, *rest
):
    (p_refs, (wx_ref, wa_ref, bn1_ref, an1_ref, wn2_ref, bn2_ref, aact_ref,
              h_ref, st_ref)) = rest[:k], rest[k:]
    i = pl.program_id(0)
    ag = p_refs[0][0]
    for pr in p_refs[1:]:
        ag = ag + pr[0]
    t = (
        jnp.dot(x_ref[...], wx_ref[...], preferred_element_type=_F32)
        + jnp.dot(ag, wa_ref[...], preferred_element_type=_F32)
        + bn1_ref[...]
    )
    h1 = jnp.where(t >= 0.0, t, an1_ref[...] * t)
    y = jnp.dot(h1, wn2_ref[...], preferred_element_type=_F32) + bn2_ref[...]
    h2 = jnp.where(y >= 0.0, y, aact_ref[...] * y)
    h_ref[...] = h2

    @pl.when(i == 0)
    def _():
        st_ref[...] = jnp.zeros_like(st_ref)

    st_ref[0:1, :] += jnp.sum(h2, axis=0, keepdims=True)
    st_ref[1:2, :] += jnp.sum(h2 * h2, axis=0, keepdims=True)


def _node_call(x, pflats, wx, wa, bn1row, an1row, wn2, bn2row, aactrow, bn):
    n, d = x.shape
    h = wx.shape[1]
    out = wn2.shape[1]
    k = len(pflats) * _NC
    grid = n // bn
    return pl.pallas_call(
        functools.partial(_node_body, k),
        grid=(grid,),
        in_specs=[
            pl.BlockSpec((bn, d), lambda i: (i, 0)),
        ] + [
            pl.BlockSpec((1, bn, h), lambda i, cc=cc: (cc, i, 0))
            for _ in range(len(pflats)) for cc in range(_NC)
        ] + [
            pl.BlockSpec((d, h), lambda i: (0, 0)),
            pl.BlockSpec((h, h), lambda i: (0, 0)),
            pl.BlockSpec((1, h), lambda i: (0, 0)),
            pl.BlockSpec((1, h), lambda i: (0, 0)),
            pl.BlockSpec((h, out), lambda i: (0, 0)),
            pl.BlockSpec((1, out), lambda i: (0, 0)),
            pl.BlockSpec((1, out), lambda i: (0, 0)),
        ],
        out_specs=[
            pl.BlockSpec((bn, out), lambda i: (i, 0)),
            pl.BlockSpec((8, out), lambda i: (0, 0)),
        ],
        out_shape=[
            jax.ShapeDtypeStruct((n, out), _F32),
            jax.ShapeDtypeStruct((8, out), _F32),
        ],
    )(x, *[pf for pf in pflats for _ in range(_NC)], wx, wa, bn1row,
      an1row, wn2, bn2row, aactrow)


def _bn_body(n, h_ref, st_ref, g_ref, b_ref, o_ref):
    s = st_ref[0:1, :]
    ss = st_ref[1:2, :]
    mean = s / n
    var = ss / n - mean * mean
    inv = lax.rsqrt(var + 1e-5)
    o_ref[...] = (h_ref[...] - mean) * (inv * g_ref[...]) + b_ref[...]


def _bn_call(h2, stats, grow, brow, bn):
    n, out = h2.shape
    grid = n // bn
    return pl.pallas_call(
        functools.partial(_bn_body, float(n)),
        grid=(grid,),
        in_specs=[
            pl.BlockSpec((bn, out), lambda i: (i, 0)),
            pl.BlockSpec((8, out), lambda i: (0, 0)),
            pl.BlockSpec((1, out), lambda i: (0, 0)),
            pl.BlockSpec((1, out), lambda i: (0, 0)),
        ],
        out_specs=pl.BlockSpec((bn, out), lambda i: (i, 0)),
        out_shape=jax.ShapeDtypeStruct((n, out), _F32),
    )(h2, stats, grow, brow)


def kernel(
    x, edge_index, edge_attr, W1, b1, a1, W2, b2, a2, Wn1, bn1, an1, Wn2,
    bn2, a_act, gamma, beta
):
    n, d = x.shape
    e = edge_index.shape[1]
    ed = edge_attr.shape[1]
    h = W2.shape[0]
    out = Wn2.shape[1]
    assert e % _SUB == 0
    nsub = e // _SUB

    b1row = b1.reshape(1, h)
    b2row = b2.reshape(1, h)
    bn1row = bn1.reshape(1, h)
    bn2row = bn2.reshape(1, out)
    a1row = jnp.broadcast_to(a1.reshape(1, 1), (1, h))
    a2row = jnp.broadcast_to(a2.reshape(1, 1), (1, h))
    an1row = jnp.broadcast_to(an1.reshape(1, 1), (1, h))
    aactrow = jnp.broadcast_to(a_act.reshape(1, 1), (1, out))
    grow = gamma.reshape(1, out)
    brow = beta.reshape(1, out)

    W1i = W1[:d]
    W1j = W1[d : 2 * d]
    W1e = W1[2 * d :]

    pA = [0] * h
    for jg in range(h // 32):
        for i in range(16):
            pA[32 * jg + 2 * i] = 32 * jg + i
            pA[32 * jg + 2 * i + 1] = 32 * jg + 16 + i
    pF = [pA[2 * c] for c in range(h // 2)]
    pF += [pA[2 * c + 1] for c in range(h // 2)]
    W1e = W1e[:, pF]
    W2p = W2[pF, :]

    p, q = _pq_call(x, W1i, W1j, b1row, bn=2000)

    n_slices = 2
    es = e // n_slices
    nsub_s = es // _SUB
    nsub_pad = -(-nsub_s // _NW) * _NW
    ei3 = edge_index.reshape(2, nsub, 1, _SUB)
    plist = []
    for s in range(n_slices):
        ei4p = jnp.pad(
            ei3[:, s * nsub_s : (s + 1) * nsub_s],
            ((0, 0), (0, nsub_pad - nsub_s), (0, 0), (0, 0)),
        )
        g = _gather_call(p, q, ei4p, nsub_s)
        m2 = _edge_call(
            g.reshape(es, h // 2), edge_attr[s * es : (s + 1) * es], W1e,
            W2p, b2row, a1row, a2row, be=2000
        )
        pt = _scatter_call(m2.reshape(nsub_s, _SUB, h), ei4p, n)
        plist.append(pt.reshape(_NC, -1, h))
    h2, stats = _node_call(
        x, plist, Wn1[:d], Wn1[d:], bn1row, an1row, Wn2,
        bn2row, aactrow, bn=2000
    )
    return _bn_call(h2, stats, grow, brow, bn=2000)


# speedup vs baseline: 1.0303x; 1.0303x over previous
import functools

import jax
import jax.numpy as jnp
from jax import lax
from jax.experimental import pallas as pl
from jax.experimental.pallas import tpu as pltpu
from jax.experimental.pallas import tpu_sc as plsc

_F32 = jnp.float32

_NC = 2
_NS = 16
_NW = _NC * _NS
_SUB = 128


def _pq_body(x_ref, wi_ref, wj_ref, b1_ref, p_ref, q_ref):
    xv = x_ref[...]
    p_ref[...] = (
        jnp.dot(xv, wi_ref[...], preferred_element_type=_F32) + b1_ref[...]
    )
    q_ref[...] = jnp.dot(xv, wj_ref[...], preferred_element_type=_F32)


def _pq_call(x, wi, wj, b1row, bn):
    n, d = x.shape
    h = wi.shape[1]
    grid = n // bn
    return pl.pallas_call(
        _pq_body,
        grid=(grid,),
        in_specs=[
            pl.BlockSpec((bn, d), lambda i: (i, 0)),
            pl.BlockSpec((d, h), lambda i: (0, 0)),
            pl.BlockSpec((d, h), lambda i: (0, 0)),
            pl.BlockSpec((1, h), lambda i: (0, 0)),
        ],
        out_specs=[
            pl.BlockSpec((bn, h), lambda i: (i, 0)),
            pl.BlockSpec((bn, h), lambda i: (i, 0)),
        ],
        out_shape=[
            jax.ShapeDtypeStruct((n, h), _F32),
            jax.ShapeDtypeStruct((n, h), _F32),
        ],
    )(x, wi, wj, b1row)


def _gather_call(p, q, ei4p, nsub):
    n, h = p.shape
    nsub_pad = ei4p.shape[1]
    cpw = nsub_pad // _NW
    mesh = plsc.VectorSubcoreMesh(core_axis_name="c", subcore_axis_name="s")

    @functools.partial(
        pl.kernel,
        out_type=jax.ShapeDtypeStruct((nsub, _SUB, h), _F32),
        mesh=mesh,
        scratch_types=[
            pltpu.VMEM((cpw, 1, _SUB), jnp.int32),
            pltpu.VMEM((cpw, 1, _SUB), jnp.int32),
            pltpu.VMEM((_SUB, h), _F32),
            pltpu.VMEM((_SUB, h), _F32),
            pltpu.VMEM((_SUB, h), _F32),
            pltpu.VMEM((_SUB, h), _F32),
            pltpu.SemaphoreType.DMA,
            pltpu.SemaphoreType.DMA,
        ],
    )
    def gather_k(
        p_hbm, q_hbm, ei_hbm, g_hbm, idxd_all, idxs_all, bufp0, bufq0,
        bufp1, bufq1, sem0, sem1
    ):
        wid = lax.axis_index("s") * _NC + lax.axis_index("c")
        w0 = wid * cpw
        nm = jnp.minimum(cpw, nsub - w0)

        pltpu.sync_copy(ei_hbm.at[1, pl.ds(w0, cpw)], idxd_all)
        pltpu.sync_copy(ei_hbm.at[0, pl.ds(w0, cpw)], idxs_all)

        bufs = ((bufp0, bufq0, sem0), (bufp1, bufq1, sem1))

        def issue(c, bp, bq, sem):
            @pl.when(c < nm)
            def _():
                pltpu.async_copy(p_hbm.at[idxd_all.at[c, 0]], bp, sem)
                pltpu.async_copy(q_hbm.at[idxs_all.at[c, 0]], bq, sem)

        def process(c, bp, bq, sem):
            @pl.when(c < nm)
            def _():
                pltpu.make_async_copy(
                    p_hbm.at[idxd_all.at[c, 0]], bp, sem
                ).wait()
                pltpu.make_async_copy(
                    q_hbm.at[idxs_all.at[c, 0]], bq, sem
                ).wait()

                def add_row(i, acc):
                    for j in range(h // 16):
                        sl = pl.ds(j * 16, 16)
                        bp[i, sl] = bp[i, sl] + bq[i, sl]
                    return acc

                lax.fori_loop(0, _SUB, add_row, 0)
                pltpu.sync_copy(bp, g_hbm.at[w0 + c])

        issue(0, *bufs[0])
        issue(1, *bufs[1])

        def body(k, carry):
            c0 = k * 2
            for b in range(2):
                bp, bq, sem = bufs[b]
                process(c0 + b, bp, bq, sem)
                issue(c0 + b + 2, bp, bq, sem)
            return carry

        lax.fori_loop(0, (cpw + 1) // 2, body, 0)

    return gather_k(p, q, ei4p)


def _edge_body(g_ref, ea_ref, we_ref, w2_ref, b2_ref, a1_ref, a2_ref, o_ref):
    z = g_ref[...] + jnp.dot(
        ea_ref[...], we_ref[...], preferred_element_type=_F32
    )
    m1 = jnp.where(z >= 0.0, z, a1_ref[...] * z)
    y = jnp.dot(m1, w2_ref[...], preferred_element_type=_F32) + b2_ref[...]
    o_ref[...] = jnp.where(y >= 0.0, y, a2_ref[...] * y)


def _edge_call(g, ea, we, w2, b2row, a1row, a2row, be):
    e, h = g.shape
    ed = ea.shape[1]
    grid = e // be
    return pl.pallas_call(
        _edge_body,
        grid=(grid,),
        in_specs=[
            pl.BlockSpec((be, h), lambda i: (i, 0)),
            pl.BlockSpec((be, ed), lambda i: (i, 0)),
            pl.BlockSpec((ed, h), lambda i: (0, 0)),
            pl.BlockSpec((h, h), lambda i: (0, 0)),
            pl.BlockSpec((1, h), lambda i: (0, 0)),
            pl.BlockSpec((1, h), lambda i: (0, 0)),
            pl.BlockSpec((1, h), lambda i: (0, 0)),
        ],
        out_specs=pl.BlockSpec((be, h), lambda i: (i, 0)),
        out_shape=jax.ShapeDtypeStruct((e, h), _F32),
    )(g, ea, we, w2, b2row, a1row, a2row)


def _scatter_call(m3, ei4p, n):
    nsub, sub, h = m3.shape
    nsub_pad = ei4p.shape[1]
    cpw = nsub_pad // _NW
    n_chunks = -(-n // _SUB)
    n_pad = n_chunks * _SUB
    cpt = -(-n_chunks // _NS)
    mesh = plsc.VectorSubcoreMesh(core_axis_name="c", subcore_axis_name="s")

    @functools.partial(
        pl.kernel,
        out_type=jax.ShapeDtypeStruct((_NC, n_chunks, _SUB, h), _F32),
        mesh=mesh,
        scratch_types=[
            pltpu.VMEM((cpw, 1, _SUB), jnp.int32),
            pltpu.VMEM((_SUB, h), _F32),
            pltpu.VMEM((_SUB, h), _F32),
            pltpu.VMEM_SHARED((n_pad, h), _F32),
            pltpu.SemaphoreType.DMA,
            pltpu.SemaphoreType.DMA,
        ],
    )
    def scatter_k(m_hbm, ei_hbm, out_hbm, idxd_all, bufm0, bufm1, acc,
                  sem0, sem1):
        cid = lax.axis_index("c")
        sid = lax.axis_index("s")
        wid = sid * _NC + cid
        w0 = wid * cpw
        nm = jnp.minimum(cpw, nsub - w0)

        pltpu.sync_copy(ei_hbm.at[1, pl.ds(w0, cpw)], idxd_all)

        def zrow(i, acc_c):
            for j in range(h // 16):
                bufm0[i, pl.ds(j * 16, 16)] = jnp.zeros((16,), _F32)
            return acc_c

        lax.fori_loop(0, _SUB, zrow, 0)
        for c in range(cpt):
            ck = c * _NS + sid

            @pl.when(ck < n_chunks)
            def _():
                row0 = pl.multiple_of(ck * _SUB, 8)
                pltpu.sync_copy(bufm0, acc.at[pl.ds(row0, _SUB)])

        plsc.subcore_barrier()

        bufs = ((bufm0, sem0), (bufm1, sem1))

        def issue(c, bm, sem):
            @pl.when(c < nm)
            def _():
                pltpu.async_copy(m_hbm.at[w0 + c], bm, sem)

        def process(c, bm, sem):
            @pl.when(c < nm)
            def _():
                pltpu.make_async_copy(m_hbm.at[w0 + c], bm, sem).wait()
                pltpu.sync_copy(bm, acc.at[idxd_all.at[c, 0]], add=True)

        issue(0, *bufs[0])
        issue(1, *bufs[1])

        def body(k, carry):
            c0 = k * 2
            for b in range(2):
                bm, sem = bufs[b]
                process(c0 + b, bm, sem)
                issue(c0 + b + 2, bm, sem)
            return carry

        lax.fori_loop(0, (cpw + 1) // 2, body, 0)
        plsc.subcore_barrier()

        for c in range(cpt):
            ck = c * _NS + sid

            @pl.when(ck < n_chunks)
            def _():
                row0 = pl.multiple_of(ck * _SUB, 8)
                pltpu.sync_copy(acc.at[pl.ds(row0, _SUB)], bufm0)
                pltpu.sync_copy(bufm0, out_hbm.at[cid, ck])

    return scatter_k(m3, ei4p)


def _node_body(
    k, x_ref, *rest
):
    (p_refs, (wx_ref, wa_ref, bn1_ref, an1_ref, wn2_ref, bn2_ref, aact_ref,
              h_ref, st_ref)) = rest[:k], rest[k:]
    i = pl.program_id(0)
    ag = p_refs[0][0]
    for pr in p_refs[1:]:
        ag = ag + pr[0]
    t = (
        jnp.dot(x_ref[...], wx_ref[...], preferred_element_type=_F32)
        + jnp.dot(ag, wa_ref[...], preferred_element_type=_F32)
        + bn1_ref[...]
    )
    h1 = jnp.where(t >= 0.0, t, an1_ref[...] * t)
    y = jnp.dot(h1, wn2_ref[...], preferred_element_type=_F32) + bn2_ref[...]
    h2 = jnp.where(y >= 0.0, y, aact_ref[...] * y)
    h_ref[...] = h2

    @pl.when(i == 0)
    def _():
        st_ref[...] = jnp.zeros_like(st_ref)

    st_ref[0:1, :] += jnp.sum(h2, axis=0, keepdims=True)
    st_ref[1:2, :] += jnp.sum(h2 * h2, axis=0, keepdims=True)


def _node_call(x, pflats, wx, wa, bn1row, an1row, wn2, bn2row, aactrow, bn):
    n, d = x.shape
    h = wx.shape[1]
    out = wn2.shape[1]
    k = len(pflats) * _NC
    grid = n // bn
    return pl.pallas_call(
        functools.partial(_node_body, k),
        grid=(grid,),
        in_specs=[
            pl.BlockSpec((bn, d), lambda i: (i, 0)),
        ] + [
            pl.BlockSpec((1, bn, h), lambda i, cc=cc: (cc, i, 0))
            for _ in range(len(pflats)) for cc in range(_NC)
        ] + [
            pl.BlockSpec((d, h), lambda i: (0, 0)),
            pl.BlockSpec((h, h), lambda i: (0, 0)),
            pl.BlockSpec((1, h), lambda i: (0, 0)),
            pl.BlockSpec((1, h), lambda i: (0, 0)),
            pl.BlockSpec((h, out), lambda i: (0, 0)),
            pl.BlockSpec((1, out), lambda i: (0, 0)),
            pl.BlockSpec((1, out), lambda i: (0, 0)),
        ],
        out_specs=[
            pl.BlockSpec((bn, out), lambda i: (i, 0)),
            pl.BlockSpec((8, out), lambda i: (0, 0)),
        ],
        out_shape=[
            jax.ShapeDtypeStruct((n, out), _F32),
            jax.ShapeDtypeStruct((8, out), _F32),
        ],
    )(x, *[pf for pf in pflats for _ in range(_NC)], wx, wa, bn1row,
      an1row, wn2, bn2row, aactrow)


def _bn_body(n, h_ref, st_ref, g_ref, b_ref, o_ref):
    s = st_ref[0:1, :]
    ss = st_ref[1:2, :]
    mean = s / n
    var = ss / n - mean * mean
    inv = lax.rsqrt(var + 1e-5)
    o_ref[...] = (h_ref[...] - mean) * (inv * g_ref[...]) + b_ref[...]


def _bn_call(h2, stats, grow, brow, bn):
    n, out = h2.shape
    grid = n // bn
    return pl.pallas_call(
        functools.partial(_bn_body, float(n)),
        grid=(grid,),
        in_specs=[
            pl.BlockSpec((bn, out), lambda i: (i, 0)),
            pl.BlockSpec((8, out), lambda i: (0, 0)),
            pl.BlockSpec((1, out), lambda i: (0, 0)),
            pl.BlockSpec((1, out), lambda i: (0, 0)),
        ],
        out_specs=pl.BlockSpec((bn, out), lambda i: (i, 0)),
        out_shape=jax.ShapeDtypeStruct((n, out), _F32),
    )(h2, stats, grow, brow)


def kernel(
    x, edge_index, edge_attr, W1, b1, a1, W2, b2, a2, Wn1, bn1, an1, Wn2,
    bn2, a_act, gamma, beta
):
    n, d = x.shape
    e = edge_index.shape[1]
    ed = edge_attr.shape[1]
    h = W2.shape[0]
    out = Wn2.shape[1]
    assert e % _SUB == 0
    nsub = e // _SUB

    b1row = b1.reshape(1, h)
    b2row = b2.reshape(1, h)
    bn1row = bn1.reshape(1, h)
    bn2row = bn2.reshape(1, out)
    a1row = jnp.broadcast_to(a1.reshape(1, 1), (1, h))
    a2row = jnp.broadcast_to(a2.reshape(1, 1), (1, h))
    an1row = jnp.broadcast_to(an1.reshape(1, 1), (1, h))
    aactrow = jnp.broadcast_to(a_act.reshape(1, 1), (1, out))
    grow = gamma.reshape(1, out)
    brow = beta.reshape(1, out)

    W1i = W1[:d]
    W1j = W1[d : 2 * d]
    W1e = W1[2 * d :]

    W2p = W2

    p, q = _pq_call(x, W1i, W1j, b1row, bn=2000)

    n_slices = 4
    es = e // n_slices
    nsub_s = es // _SUB
    nsub_pad = -(-nsub_s // _NW) * _NW
    ei3 = edge_index.reshape(2, nsub, 1, _SUB)
    plist = []
    for s in range(n_slices):
        ei4p = jnp.pad(
            ei3[:, s * nsub_s : (s + 1) * nsub_s],
            ((0, 0), (0, nsub_pad - nsub_s), (0, 0), (0, 0)),
        )
        g = _gather_call(p, q, ei4p, nsub_s)
        m2 = _edge_call(
            g.reshape(es, h), edge_attr[s * es : (s + 1) * es], W1e,
            W2p, b2row, a1row, a2row, be=2000
        )
        pt = _scatter_call(m2.reshape(nsub_s, _SUB, h), ei4p, n)
        plist.append(pt.reshape(_NC, -1, h))
    h2, stats = _node_call(
        x, plist, Wn1[:d], Wn1[d:], bn1row, an1row, Wn2,
        bn2row, aactrow, bn=2000
    )
    return _bn_call(h2, stats, grow, brow, bn=2000)
